# baseline (speedup 1.0000x reference)
import functools

import jax
import jax.numpy as jnp
from jax import lax
from jax.experimental import pallas as pl
from jax.experimental.pallas import tpu as pltpu
from jax.experimental.pallas import tpu_sc as plsc

N = 10000
E = 320000
D = 128

NC = 2
NS = 16
NW = NC * NS
EPT = E // NW
C = 80
NCH = EPT // C
NB = 4
NBI = 5
NP = 10240
RPT = NP // NS

_SLOPE = 0.25
_BM = 2000

_GDN = lax.GatherDimensionNumbers(
    offset_dims=(), collapsed_slice_dims=(0,), start_index_map=(0,))


def _leaky(v):
    return jnp.where(v >= 0, v, _SLOPE * v)


def _lane_bcast(vec16, i):
    idx = jnp.full((16, 1), i, jnp.int32)
    return lax.gather(vec16, idx, _GDN, (1,),
                      mode=lax.GatherScatterMode.PROMISE_IN_BOUNDS)


def _mm_body(x_ref, w_ref, o_ref):
    o_ref[...] = jnp.dot(x_ref[...], w_ref[...],
                         preferred_element_type=jnp.float32)


def _fused_body(p_ref, b_ref, w_ref, o_ref):
    h = _leaky(p_ref[0] + p_ref[1] + b_ref[...])
    o_ref[...] = jnp.dot(h, w_ref[...], preferred_element_type=jnp.float32)


def _final_body(p_ref, b_ref, o_ref):
    o_ref[...] = _leaky(p_ref[0] + p_ref[1] + b_ref[...])


def _mm(x, w):
    return pl.pallas_call(
        _mm_body,
        grid=(N // _BM,),
        in_specs=[
            pl.BlockSpec((_BM, x.shape[1]), lambda i: (i, 0)),
            pl.BlockSpec(w.shape, lambda i: (0, 0)),
        ],
        out_specs=pl.BlockSpec((_BM, w.shape[1]), lambda i: (i, 0)),
        out_shape=jax.ShapeDtypeStruct((N, w.shape[1]), jnp.float32),
    )(x, w)


def _fused_mm(p, b, w):
    return pl.pallas_call(
        _fused_body,
        grid=(N // _BM,),
        in_specs=[
            pl.BlockSpec((2, _BM, D), lambda i: (0, i, 0)),
            pl.BlockSpec((1, D), lambda i: (0, 0)),
            pl.BlockSpec(w.shape, lambda i: (0, 0)),
        ],
        out_specs=pl.BlockSpec((_BM, w.shape[1]), lambda i: (i, 0)),
        out_shape=jax.ShapeDtypeStruct((N, w.shape[1]), jnp.float32),
    )(p, b, w)


def _final(p, b):
    return pl.pallas_call(
        _final_body,
        grid=(N // _BM,),
        in_specs=[
            pl.BlockSpec((2, _BM, D), lambda i: (0, i, 0)),
            pl.BlockSpec((1, D), lambda i: (0, 0)),
        ],
        out_specs=pl.BlockSpec((_BM, D), lambda i: (i, 0)),
        out_shape=jax.ShapeDtypeStruct((N, D), jnp.float32),
    )(p, b)


_mesh = plsc.VectorSubcoreMesh(core_axis_name="c", subcore_axis_name="s",
                               num_cores=NC, num_subcores=NS)


@functools.partial(
    pl.kernel,
    out_type=jax.ShapeDtypeStruct((2, NP, D), jnp.float32),
    mesh=_mesh,
    scratch_types=[
        pltpu.VMEM((NBI, C), jnp.int32),
        pltpu.VMEM((NBI, C), jnp.int32),
        pltpu.VMEM((NBI, C), jnp.float32),
        pltpu.VMEM((NB, C, D), jnp.float32),
        pltpu.VMEM_SHARED((NP, D), jnp.float32),
        pltpu.SemaphoreType.DMA,
        pltpu.SemaphoreType.DMA,
        pltpu.SemaphoreType.DMA,
    ],
)
def _agg(support_hbm, col_hbm, row_hbm, w_hbm, zeros_hbm, out_hbm,
         colv, rowv, wv, rows, acc, isem, gsem, ssem):
    cid = lax.axis_index("c")
    sid = lax.axis_index("s")
    wid = cid * NS + sid

    pltpu.sync_copy(zeros_hbm, acc.at[pl.ds(sid * RPT, RPT)])
    plsc.subcore_barrier()

    def idx_start(cj, k):
        pltpu.async_copy(col_hbm.at[wid, cj], colv.at[k], isem)
        pltpu.async_copy(row_hbm.at[wid, cj], rowv.at[k], isem)
        pltpu.async_copy(w_hbm.at[wid, cj], wv.at[k], isem)

    def idx_wait(cj, k):
        pltpu.make_async_copy(col_hbm.at[wid, cj], colv.at[k], isem).wait()
        pltpu.make_async_copy(row_hbm.at[wid, cj], rowv.at[k], isem).wait()
        pltpu.make_async_copy(w_hbm.at[wid, cj], wv.at[k], isem).wait()

    idx_start(0, 0)
    idx_start(1, 1)
    idx_start(2, 2)
    idx_start(3, 3)
    idx_wait(0, 0)
    pltpu.async_copy(support_hbm.at[colv.at[0]], rows.at[0], gsem)
    idx_wait(1, 1)
    pltpu.async_copy(support_hbm.at[colv.at[1]], rows.at[1], gsem)
    idx_wait(2, 2)
    pltpu.async_copy(support_hbm.at[colv.at[2]], rows.at[2], gsem)

    def chunk_body(cj, carry2):
        b = lax.rem(cj, NB)
        bn3 = lax.rem(cj + 3, NB)
        k = lax.rem(cj, NBI)
        kn3 = lax.rem(cj + 3, NBI)
        kn4 = lax.rem(cj + 4, NBI)

        @pl.when(cj >= 1)
        def _():
            pltpu.make_async_copy(rows.at[bn3],
                                  acc.at[rowv.at[kn4]], ssem).wait()

        @pl.when(cj < NCH - 4)
        def _():
            idx_start(cj + 4, kn4)

        @pl.when(cj < NCH - 3)
        def _():
            idx_wait(cj + 3, kn3)
            pltpu.async_copy(support_hbm.at[colv.at[kn3]],
                             rows.at[bn3], gsem)

        pltpu.make_async_copy(support_hbm.at[colv.at[k]],
                              rows.at[b], gsem).wait()

        for g in range(C // 16):
            w16 = wv[k, pl.ds(g * 16, 16)]
            for i in range(16):
                wsp = _lane_bcast(w16, i)
                j = g * 16 + i
                for cb in range(D // 16):
                    sl = pl.ds(cb * 16, 16)
                    rows[b, j, sl] = rows[b, j, sl] * wsp

        pltpu.async_copy(rows.at[b], acc.at[rowv.at[k]], ssem, add=True)
        return carry2

    lax.fori_loop(0, NCH, chunk_body, 0)

    pltpu.make_async_copy(rows.at[(NCH - 1) % NB],
                          acc.at[rowv.at[(NCH - 1) % NBI]], ssem).wait()

    plsc.subcore_barrier()

    pltpu.sync_copy(acc.at[pl.ds(sid * RPT, RPT)],
                    out_hbm.at[cid, pl.ds(sid * RPT, RPT)])


def kernel(x, edge_index, edge_weight, W1, b1, W2, b2, W3, b3):
    col4 = edge_index[1].reshape(NW, NCH, C)
    row4 = edge_index[0].reshape(NW, NCH, C)
    w4 = edge_weight.reshape(NW, NCH, C)
    zeros = jnp.zeros((RPT, D), jnp.float32)
    b1r = b1.reshape(1, D)
    b2r = b2.reshape(1, D)
    b3r = b3.reshape(1, D)

    s1 = _mm(x, W1)
    p1 = _agg(s1, col4, row4, w4, zeros)
    s2 = _fused_mm(p1, b1r, W2)
    p2 = _agg(s2, col4, row4, w4, zeros)
    s3 = _fused_mm(p2, b2r, W3)
    p3 = _agg(s3, col4, row4, w4, zeros)
    return _final(p3, b3r)


# speedup vs baseline: 1.0527x; 1.0527x over previous
import functools

import jax
import jax.numpy as jnp
from jax import lax
from jax.experimental import pallas as pl
from jax.experimental.pallas import tpu as pltpu
from jax.experimental.pallas import tpu_sc as plsc

N = 10000
E = 320000
D = 128

NC = 2
NS = 16
NW = NC * NS
EPT = E // NW
C = 80
NCH = EPT // C
NB = 4
NBI = 5
NP = 10240
RPT = NP // NS

_SLOPE = 0.25
_BM = 2000

_GDN = lax.GatherDimensionNumbers(
    offset_dims=(), collapsed_slice_dims=(0,), start_index_map=(0,))


def _leaky(v):
    return jnp.where(v >= 0, v, _SLOPE * v)


def _lane_bcast(vec16, i):
    idx = jnp.full((16, 1), i, jnp.int32)
    return lax.gather(vec16, idx, _GDN, (1,),
                      mode=lax.GatherScatterMode.PROMISE_IN_BOUNDS)


def _mm_body(x_ref, w_ref, o_ref):
    o_ref[...] = jnp.dot(x_ref[...], w_ref[...],
                         preferred_element_type=jnp.float32)


def _fused_body(p_ref, b_ref, w_ref, o_ref):
    h = _leaky(p_ref[0] + p_ref[1] + b_ref[...])
    o_ref[...] = jnp.dot(h, w_ref[...], preferred_element_type=jnp.float32)


def _final_body(p_ref, b_ref, o_ref):
    o_ref[...] = _leaky(p_ref[0] + p_ref[1] + b_ref[...])


def _mm(x, w):
    return pl.pallas_call(
        _mm_body,
        grid=(N // _BM,),
        in_specs=[
            pl.BlockSpec((_BM, x.shape[1]), lambda i: (i, 0)),
            pl.BlockSpec(w.shape, lambda i: (0, 0)),
        ],
        out_specs=pl.BlockSpec((_BM, w.shape[1]), lambda i: (i, 0)),
        out_shape=jax.ShapeDtypeStruct((N, w.shape[1]), jnp.float32),
    )(x, w)


def _fused_mm(p, b, w):
    return pl.pallas_call(
        _fused_body,
        grid=(N // _BM,),
        in_specs=[
            pl.BlockSpec((2, _BM, D), lambda i: (0, i, 0)),
            pl.BlockSpec((1, D), lambda i: (0, 0)),
            pl.BlockSpec(w.shape, lambda i: (0, 0)),
        ],
        out_specs=pl.BlockSpec((_BM, w.shape[1]), lambda i: (i, 0)),
        out_shape=jax.ShapeDtypeStruct((N, w.shape[1]), jnp.float32),
    )(p, b, w)


def _final(p, b):
    return pl.pallas_call(
        _final_body,
        grid=(N // _BM,),
        in_specs=[
            pl.BlockSpec((2, _BM, D), lambda i: (0, i, 0)),
            pl.BlockSpec((1, D), lambda i: (0, 0)),
        ],
        out_specs=pl.BlockSpec((_BM, D), lambda i: (i, 0)),
        out_shape=jax.ShapeDtypeStruct((N, D), jnp.float32),
    )(p, b)


_mesh = plsc.VectorSubcoreMesh(core_axis_name="c", subcore_axis_name="s",
                               num_cores=NC, num_subcores=NS)


@functools.partial(
    pl.kernel,
    out_type=jax.ShapeDtypeStruct((2, NP, D), jnp.float32),
    mesh=_mesh,
    scratch_types=[
        pltpu.VMEM((NBI, C), jnp.int32),
        pltpu.VMEM((NBI, C), jnp.int32),
        pltpu.VMEM((NBI, C), jnp.float32),
        pltpu.VMEM((NB, C, D), jnp.float32),
        pltpu.VMEM_SHARED((NP, D), jnp.float32),
        pltpu.SemaphoreType.DMA,
        pltpu.SemaphoreType.DMA,
        pltpu.SemaphoreType.DMA,
    ],
)
def _agg(support_hbm, col_hbm, row_hbm, w_hbm, zeros_hbm, out_hbm,
         colv, rowv, wv, rows, acc, isem, gsem, ssem):
    cid = lax.axis_index("c")
    sid = lax.axis_index("s")
    wid = cid * NS + sid

    pltpu.sync_copy(zeros_hbm, acc.at[pl.ds(sid * RPT, RPT)])
    plsc.subcore_barrier()

    def idx_start(cj, k):
        pltpu.async_copy(col_hbm.at[wid, cj], colv.at[k], isem)
        pltpu.async_copy(row_hbm.at[wid, cj], rowv.at[k], isem)
        pltpu.async_copy(w_hbm.at[wid, cj], wv.at[k], isem)

    def idx_wait(cj, k):
        pltpu.make_async_copy(col_hbm.at[wid, cj], colv.at[k], isem).wait()
        pltpu.make_async_copy(row_hbm.at[wid, cj], rowv.at[k], isem).wait()
        pltpu.make_async_copy(w_hbm.at[wid, cj], wv.at[k], isem).wait()

    idx_start(0, 0)
    idx_start(1, 1)
    idx_start(2, 2)
    idx_wait(0, 0)
    pltpu.async_copy(support_hbm.at[colv.at[0]], rows.at[0], gsem)
    idx_wait(1, 1)
    pltpu.async_copy(support_hbm.at[colv.at[1]], rows.at[1], gsem)

    def chunk_body(cj, carry2):
        b = lax.rem(cj, NB)
        bn2 = lax.rem(cj + 2, NB)
        k = lax.rem(cj, NBI)
        kn2 = lax.rem(cj + 2, NBI)
        kn3 = lax.rem(cj + 3, NBI)

        @pl.when(cj >= 2)
        def _():
            pltpu.make_async_copy(rows.at[bn2],
                                  acc.at[rowv.at[kn3]], ssem).wait()

        @pl.when(cj < NCH - 3)
        def _():
            idx_start(cj + 3, kn3)

        @pl.when(cj < NCH - 2)
        def _():
            idx_wait(cj + 2, kn2)
            pltpu.async_copy(support_hbm.at[colv.at[kn2]],
                             rows.at[bn2], gsem)

        pltpu.make_async_copy(support_hbm.at[colv.at[k]],
                              rows.at[b], gsem).wait()

        for g in range(C // 16):
            w16 = wv[k, pl.ds(g * 16, 16)]
            for i in range(16):
                wsp = jnp.full((16,), w16[i], jnp.float32)
                j = g * 16 + i
                for cb in range(D // 16):
                    sl = pl.ds(cb * 16, 16)
                    rows[b, j, sl] = rows[b, j, sl] * wsp

        pltpu.async_copy(rows.at[b], acc.at[rowv.at[k]], ssem, add=True)
        return carry2

    lax.fori_loop(0, NCH, chunk_body, 0)

    pltpu.make_async_copy(rows.at[(NCH - 2) % NB],
                          acc.at[rowv.at[(NCH - 2) % NBI]], ssem).wait()
    pltpu.make_async_copy(rows.at[(NCH - 1) % NB],
                          acc.at[rowv.at[(NCH - 1) % NBI]], ssem).wait()

    plsc.subcore_barrier()

    pltpu.sync_copy(acc.at[pl.ds(sid * RPT, RPT)],
                    out_hbm.at[cid, pl.ds(sid * RPT, RPT)])


def kernel(x, edge_index, edge_weight, W1, b1, W2, b2, W3, b3):
    col4 = edge_index[1].reshape(NW, NCH, C)
    row4 = edge_index[0].reshape(NW, NCH, C)
    w4 = edge_weight.reshape(NW, NCH, C)
    zeros = jnp.zeros((RPT, D), jnp.float32)
    b1r = b1.reshape(1, D)
    b2r = b2.reshape(1, D)
    b3r = b3.reshape(1, D)

    s1 = _mm(x, W1)
    p1 = _agg(s1, col4, row4, w4, zeros)
    s2 = _fused_mm(p1, b1r, W2)
    p2 = _agg(s2, col4, row4, w4, zeros)
    s3 = _fused_mm(p2, b2r, W3)
    p3 = _agg(s3, col4, row4, w4, zeros)
    return _final(p3, b3r)


# speedup vs baseline: 1.0595x; 1.0065x over previous
import functools

import jax
import jax.numpy as jnp
from jax import lax
from jax.experimental import pallas as pl
from jax.experimental.pallas import tpu as pltpu
from jax.experimental.pallas import tpu_sc as plsc

N = 10000
E = 320000
D = 128

NC = 2
NS = 16
NW = NC * NS
EPT = E // NW
C = 80
NCH = EPT // C
NB = 4
NBI = 5
NP = 10240
RPT = NP // NS

_SLOPE = 0.25
_BM = 2000

_GDN = lax.GatherDimensionNumbers(
    offset_dims=(), collapsed_slice_dims=(0,), start_index_map=(0,))


def _leaky(v):
    return jnp.where(v >= 0, v, _SLOPE * v)


def _lane_bcast(vec16, i):
    idx = jnp.full((16, 1), i, jnp.int32)
    return lax.gather(vec16, idx, _GDN, (1,),
                      mode=lax.GatherScatterMode.PROMISE_IN_BOUNDS)


def _mm_body(x_ref, w_ref, o_ref):
    o_ref[...] = jnp.dot(x_ref[...], w_ref[...],
                         preferred_element_type=jnp.float32)


def _fused_body(p_ref, b_ref, w_ref, o_ref):
    h = _leaky(p_ref[0] + p_ref[1] + b_ref[...])
    o_ref[...] = jnp.dot(h, w_ref[...], preferred_element_type=jnp.float32)


def _final_body(p_ref, b_ref, o_ref):
    o_ref[...] = _leaky(p_ref[0] + p_ref[1] + b_ref[...])


def _mm(x, w):
    return pl.pallas_call(
        _mm_body,
        grid=(N // _BM,),
        in_specs=[
            pl.BlockSpec((_BM, x.shape[1]), lambda i: (i, 0)),
            pl.BlockSpec(w.shape, lambda i: (0, 0)),
        ],
        out_specs=pl.BlockSpec((_BM, w.shape[1]), lambda i: (i, 0)),
        out_shape=jax.ShapeDtypeStruct((N, w.shape[1]), jnp.float32),
    )(x, w)


def _fused_mm(p, b, w):
    return pl.pallas_call(
        _fused_body,
        grid=(N // _BM,),
        in_specs=[
            pl.BlockSpec((2, _BM, D), lambda i: (0, i, 0)),
            pl.BlockSpec((1, D), lambda i: (0, 0)),
            pl.BlockSpec(w.shape, lambda i: (0, 0)),
        ],
        out_specs=pl.BlockSpec((_BM, w.shape[1]), lambda i: (i, 0)),
        out_shape=jax.ShapeDtypeStruct((N, w.shape[1]), jnp.float32),
    )(p, b, w)


def _final(p, b):
    return pl.pallas_call(
        _final_body,
        grid=(N // _BM,),
        in_specs=[
            pl.BlockSpec((2, _BM, D), lambda i: (0, i, 0)),
            pl.BlockSpec((1, D), lambda i: (0, 0)),
        ],
        out_specs=pl.BlockSpec((_BM, D), lambda i: (i, 0)),
        out_shape=jax.ShapeDtypeStruct((N, D), jnp.float32),
    )(p, b)


_mesh = plsc.VectorSubcoreMesh(core_axis_name="c", subcore_axis_name="s",
                               num_cores=NC, num_subcores=NS)


@functools.partial(
    pl.kernel,
    out_type=jax.ShapeDtypeStruct((2, NP, D), jnp.float32),
    mesh=_mesh,
    scratch_types=[
        pltpu.VMEM((NBI, C), jnp.int32),
        pltpu.VMEM((NBI, C), jnp.int32),
        pltpu.VMEM((NBI, C), jnp.float32),
        pltpu.VMEM((NB, C, D), jnp.float32),
        pltpu.VMEM_SHARED((NP, D), jnp.float32),
        pltpu.SemaphoreType.DMA,
        pltpu.SemaphoreType.DMA,
        pltpu.SemaphoreType.DMA,
    ],
)
def _agg(support_hbm, col_hbm, row_hbm, w_hbm, zeros_hbm, out_hbm,
         colv, rowv, wv, rows, acc, isem, gsem, ssem):
    cid = lax.axis_index("c")
    sid = lax.axis_index("s")
    wid = cid * NS + sid

    pltpu.sync_copy(zeros_hbm, acc.at[pl.ds(sid * RPT, RPT)])
    plsc.subcore_barrier()

    def idx_start(cj, k):
        pltpu.async_copy(col_hbm.at[wid, cj], colv.at[k], isem)
        pltpu.async_copy(row_hbm.at[wid, cj], rowv.at[k], isem)
        pltpu.async_copy(w_hbm.at[wid, cj], wv.at[k], isem)

    def idx_wait(cj, k):
        pltpu.make_async_copy(col_hbm.at[wid, cj], colv.at[k], isem).wait()
        pltpu.make_async_copy(row_hbm.at[wid, cj], rowv.at[k], isem).wait()
        pltpu.make_async_copy(w_hbm.at[wid, cj], wv.at[k], isem).wait()

    idx_start(0, 0)
    idx_start(1, 1)
    idx_start(2, 2)
    idx_wait(0, 0)
    pltpu.async_copy(support_hbm.at[colv.at[0]], rows.at[0], gsem)
    idx_wait(1, 1)
    pltpu.async_copy(support_hbm.at[colv.at[1]], rows.at[1], gsem)

    def chunk_body(cj, carry2):
        b = lax.rem(cj, NB)
        bn2 = lax.rem(cj + 2, NB)
        k = lax.rem(cj, NBI)
        kn2 = lax.rem(cj + 2, NBI)
        kn3 = lax.rem(cj + 3, NBI)

        @pl.when(cj >= 2)
        def _():
            pltpu.make_async_copy(rows.at[bn2],
                                  acc.at[rowv.at[kn3]], ssem).wait()

        @pl.when(cj < NCH - 3)
        def _():
            idx_start(cj + 3, kn3)

        @pl.when(cj < NCH - 2)
        def _():
            idx_wait(cj + 2, kn2)
            pltpu.async_copy(support_hbm.at[colv.at[kn2]],
                             rows.at[bn2], gsem)

        pltpu.make_async_copy(support_hbm.at[colv.at[k]],
                              rows.at[b], gsem).wait()

        for g in range(C // 16):
            w16 = wv[k, pl.ds(g * 16, 16)]
            for i in range(16):
                wsp = _lane_bcast(w16, i)
                j = g * 16 + i
                for cb in range(D // 16):
                    sl = pl.ds(cb * 16, 16)
                    rows[b, j, sl] = rows[b, j, sl] * wsp

        pltpu.async_copy(rows.at[b], acc.at[rowv.at[k]], ssem, add=True)
        return carry2

    lax.fori_loop(0, NCH, chunk_body, 0)

    pltpu.make_async_copy(rows.at[(NCH - 2) % NB],
                          acc.at[rowv.at[(NCH - 2) % NBI]], ssem).wait()
    pltpu.make_async_copy(rows.at[(NCH - 1) % NB],
                          acc.at[rowv.at[(NCH - 1) % NBI]], ssem).wait()

    plsc.subcore_barrier()

    pltpu.sync_copy(acc.at[pl.ds(sid * RPT, RPT)],
                    out_hbm.at[cid, pl.ds(sid * RPT, RPT)])


def kernel(x, edge_index, edge_weight, W1, b1, W2, b2, W3, b3):
    col4 = edge_index[1].reshape(NW, NCH, C)
    row4 = edge_index[0].reshape(NW, NCH, C)
    w4 = edge_weight.reshape(NW, NCH, C)
    zeros = jnp.zeros((RPT, D), jnp.float32)
    b1r = b1.reshape(1, D)
    b2r = b2.reshape(1, D)
    b3r = b3.reshape(1, D)

    s1 = _mm(x, W1)
    p1 = _agg(s1, col4, row4, w4, zeros)
    s2 = _fused_mm(p1, b1r, W2)
    p2 = _agg(s2, col4, row4, w4, zeros)
    s3 = _fused_mm(p2, b2r, W3)
    p3 = _agg(s3, col4, row4, w4, zeros)
    return _final(p3, b3r)
